# baseline (speedup 1.0000x reference)
import functools

import jax
import jax.numpy as jnp
from jax import lax
from jax.experimental import pallas as pl
from jax.experimental.pallas import tpu as pltpu
from jax.experimental.pallas import tpu_sc as plsc


def _make_pair_gather(V2, D2, B):
    info = plsc.get_sparse_core_info()
    NC, NS = info.num_cores, info.num_subcores
    NW = NC * NS
    assert B % (8 * NW) == 0
    b_per_w = B // NW
    n_chunks = b_per_w // 128
    mesh = plsc.VectorSubcoreMesh(core_axis_name="c", subcore_axis_name="s")

    @functools.partial(
        pl.kernel,
        mesh=mesh,
        out_type=jax.ShapeDtypeStruct((B, D2), jnp.float32),
        scratch_types=[
            pltpu.VMEM((n_chunks, 128), jnp.int32),
            pltpu.VMEM((b_per_w, D2), jnp.float32),
            pltpu.SemaphoreType.DMA,
        ],
    )
    def gather_kernel(table_hbm, idx_hbm, out_hbm, idx_v, rows_v, sem):
        wid = lax.axis_index("s") * NC + lax.axis_index("c")
        base = wid * b_per_w
        pltpu.sync_copy(
            idx_hbm.at[pl.ds(wid * n_chunks, n_chunks), :], idx_v
        )
        copies = []
        for j in range(n_chunks):
            copies.append(
                pltpu.make_async_copy(
                    table_hbm.at[idx_v.at[j]],
                    rows_v.at[pl.ds(j * 128, 128), :],
                    sem,
                )
            )
            copies[-1].start()
        for c in copies:
            c.wait()
        pltpu.sync_copy(rows_v, out_hbm.at[pl.ds(base, b_per_w), :])

    return gather_kernel


def kernel(entities, entity_table, relation_table):
    B = entities.shape[0]
    V, D = entity_table.shape
    table2 = entity_table.reshape(V // 2, 2 * D)
    idx = entities.astype(jnp.int32)
    idx2 = (idx >> 1).reshape(B // 128, 128)
    gather = _make_pair_gather(V // 2, 2 * D, B)
    pairs = gather(table2, idx2)
    odd = (idx & 1).astype(bool)
    return jnp.where(odd[:, None], pairs[:, D:], pairs[:, :D])
